# baseline (speedup 1.0000x reference)
import functools

import jax
import jax.numpy as jnp
from jax import lax
from jax.experimental import pallas as pl
from jax.experimental.pallas import tpu as pltpu
from jax.experimental.pallas import tpu_sc as plsc

_NC = 2
_NS = 16
_LANES = 16
_NW = _NC * _NS


def _mm1_body(x_ref, w_ref, o_ref):
    o_ref[...] = jnp.dot(x_ref[...], w_ref[...],
                         preferred_element_type=jnp.float32)


def _mm2_body(a0_ref, a1_ref, b_ref, w_ref, o_ref):
    h = jnp.maximum(a0_ref[...] + a1_ref[...] + b_ref[...], 0.0)
    o_ref[...] = jnp.dot(h, w_ref[...], preferred_element_type=jnp.float32)


def _mm3_body(a0_ref, a1_ref, b_ref, w_ref, bl_ref, o_ref):
    h = a0_ref[...] + a1_ref[...] + b_ref[...]
    o_ref[...] = (jnp.dot(h, w_ref[...], preferred_element_type=jnp.float32)
                  + bl_ref[...])


def _lsm_body(z0_ref, z1_ref, o0_ref, o1_ref):
    z0 = z0_ref[...]
    z1 = z1_ref[...]
    m = jnp.maximum(z0, z1)
    lse = m + jnp.log(jnp.exp(z0 - m) + jnp.exp(z1 - m))
    o0_ref[...] = z0 - lse
    o1_ref[...] = z1 - lse


def _tc_mm1(x, w, blk):
    n, f = x.shape
    h = w.shape[1]
    grid = n // blk
    return pl.pallas_call(
        _mm1_body,
        grid=(grid,),
        in_specs=[
            pl.BlockSpec((blk, f), lambda i: (i, 0)),
            pl.BlockSpec((f, h), lambda i: (0, 0)),
        ],
        out_specs=pl.BlockSpec((blk, h), lambda i: (i, 0)),
        out_shape=jax.ShapeDtypeStruct((n, h), jnp.float32),
    )(x, w)


def _tc_mm2(a0, a1, b_row, w, blk):
    n, f = a0.shape
    h = w.shape[1]
    grid = n // blk
    return pl.pallas_call(
        _mm2_body,
        grid=(grid,),
        in_specs=[
            pl.BlockSpec((blk, f), lambda i: (i, 0)),
            pl.BlockSpec((blk, f), lambda i: (i, 0)),
            pl.BlockSpec((1, f), lambda i: (0, 0)),
            pl.BlockSpec((f, h), lambda i: (0, 0)),
        ],
        out_specs=pl.BlockSpec((blk, h), lambda i: (i, 0)),
        out_shape=jax.ShapeDtypeStruct((n, h), jnp.float32),
    )(a0, a1, b_row, w)


def _tc_mm3(a0, a1, b_row, w8, bl8, blk):
    n, f = a0.shape
    h8 = w8.shape[1]
    grid = n // blk
    return pl.pallas_call(
        _mm3_body,
        grid=(grid,),
        in_specs=[
            pl.BlockSpec((blk, f), lambda i: (i, 0)),
            pl.BlockSpec((blk, f), lambda i: (i, 0)),
            pl.BlockSpec((1, f), lambda i: (0, 0)),
            pl.BlockSpec((f, h8), lambda i: (0, 0)),
            pl.BlockSpec((1, h8), lambda i: (0, 0)),
        ],
        out_specs=pl.BlockSpec((blk, h8), lambda i: (i, 0)),
        out_shape=jax.ShapeDtypeStruct((n, h8), jnp.float32),
    )(a0, a1, b_row, w8, bl8)


def _tc_log_softmax2(z0, z1):
    r, c = z0.shape
    return pl.pallas_call(
        _lsm_body,
        grid=(1,),
        in_specs=[
            pl.BlockSpec((r, c), lambda i: (0, 0)),
            pl.BlockSpec((r, c), lambda i: (0, 0)),
        ],
        out_specs=[
            pl.BlockSpec((r, c), lambda i: (0, 0)),
            pl.BlockSpec((r, c), lambda i: (0, 0)),
        ],
        out_shape=[
            jax.ShapeDtypeStruct((r, c), jnp.float32),
            jax.ShapeDtypeStruct((r, c), jnp.float32),
        ],
    )(z0, z1)


def _make_agg(n, f, ept, ch):
    nch = ept // ch
    noct = nch // 8
    dbase = (n // (8 * _NS)) * 8
    tail = n - _NS * dbase
    nzfull, zrem = divmod(dbase, ch)
    assert ch % _LANES == 0 and zrem % 8 == 0 and tail % 8 == 0 and tail <= ch
    assert nch == 8 * noct and noct >= 2
    nvec = f // _LANES
    mesh = plsc.VectorSubcoreMesh(core_axis_name="c", subcore_axis_name="s")

    @functools.partial(
        pl.kernel,
        out_type=jax.ShapeDtypeStruct((_NC, n, f), jnp.float32),
        mesh=mesh,
        compiler_params=pltpu.CompilerParams(needs_layout_passes=False),
        scratch_types=[
            [pltpu.VMEM((3, ch), jnp.int32)] * 8,
            [pltpu.VMEM((ch, f), jnp.float32)] * 4,
            pltpu.VMEM_SHARED((n, f), jnp.float32),
            [pltpu.SemaphoreType.DMA] * 8,
            [pltpu.SemaphoreType.DMA] * 4,
            [pltpu.SemaphoreType.DMA] * 4,
        ],
    )
    def agg(h_hbm, edges_hbm, out_hbm,
            idx_r, rows_r, acc_sh, sem_i, sem_g, sem_s):
        c = lax.axis_index("c")
        s = lax.axis_index("s")
        wid = c * _NS + s

        def _stage(k, u):
            pltpu.async_copy(edges_hbm.at[wid, k], idx_r[u], sem_i[u])

        def _wait_stage(u):
            pltpu.make_async_copy(edges_hbm.at[wid, 0], idx_r[u],
                                  sem_i[u]).wait()

        def _gather(u8, u4):
            pltpu.async_copy(h_hbm.at[idx_r[u8].at[0]], rows_r[u4],
                             sem_g[u4])

        def _wait_gather(u8, u4):
            pltpu.make_async_copy(h_hbm.at[idx_r[u8].at[0]], rows_r[u4],
                                  sem_g[u4]).wait()

        def _scale(u8, u4):
            w_v = idx_r[u8]
            rows_v = rows_r[u4]

            def _grp(g, _):
                base = g * _LANES
                wgrp = plsc.bitcast(w_v[2, pl.ds(base, _LANES)], jnp.float32)
                for j2 in range(_LANES):
                    wb = lax.gather(
                        wgrp,
                        jnp.full((_LANES, 1), j2, jnp.int32),
                        lax.GatherDimensionNumbers(
                            offset_dims=(), collapsed_slice_dims=(0,),
                            start_index_map=(0,)),
                        (1,),
                        mode=lax.GatherScatterMode.PROMISE_IN_BOUNDS)
                    for r in range(nvec):
                        rows_v[base + j2, pl.ds(r * _LANES, _LANES)] = (
                            rows_v[base + j2, pl.ds(r * _LANES, _LANES)] * wb)
                return 0
            lax.fori_loop(0, ch // _LANES, _grp, 0)

        def _scatter(u4, u8):
            pltpu.async_copy(rows_r[u4], acc_sh.at[idx_r[u8].at[1]],
                             sem_s[u4], add=True)

        def _wait_scatter(u4, u8):
            pltpu.make_async_copy(rows_r[u4], acc_sh.at[idx_r[u8].at[1]],
                                  sem_s[u4]).wait()

        def _zfill(i, _):
            for r in range(nvec):
                rows_r[0][i, pl.ds(r * _LANES, _LANES)] = jnp.zeros(
                    (_LANES,), jnp.float32)
            return 0
        lax.fori_loop(0, ch, _zfill, 0)
        for k in range(nzfull):
            pltpu.sync_copy(rows_r[0],
                            acc_sh.at[pl.ds(s * dbase + k * ch, ch)])
        if zrem:
            pltpu.sync_copy(rows_r[0].at[pl.ds(0, zrem)],
                            acc_sh.at[pl.ds(s * dbase + nzfull * ch, zrem)])

        @pl.when(s == 0)
        def _():
            pltpu.sync_copy(rows_r[0].at[pl.ds(0, tail)],
                            acc_sh.at[pl.ds(_NS * dbase, tail)])

        for k0 in range(4):
            _stage(k0, k0)
        plsc.subcore_barrier()
        _wait_stage(0)
        _gather(0, 0)

        def _oct(i, _):
            for kk in range(8):
                k = 8 * i + kk
                u4 = kk % 4
                u8 = kk
                g4 = (kk + 1) % 4
                g8 = (kk + 1) % 8
                s4 = (kk + 2) % 4
                s8 = (kk + 6) % 8
                p8 = (kk + 4) % 8
                _wait_gather(u8, u4)
                if kk == 7:
                    @pl.when(i < noct - 1)
                    def _():
                        _wait_stage(g8)
                        _gather(g8, g4)
                else:
                    _wait_stage(g8)
                    _gather(g8, g4)
                _scale(u8, u4)
                _scatter(u4, u8)
                if kk < 2:
                    @pl.when(i > 0)
                    def _():
                        _wait_scatter(s4, s8)
                else:
                    _wait_scatter(s4, s8)
                if kk >= 4:
                    @pl.when(i < noct - 1)
                    def _():
                        _stage(k + 4, p8)
                else:
                    _stage(k + 4, p8)
            return 0
        lax.fori_loop(0, noct, _oct, 0)

        for k in (nch - 2, nch - 1):
            _wait_scatter(k % 4, k % 8)

        plsc.subcore_barrier()

        pltpu.sync_copy(acc_sh.at[pl.ds(s * dbase, dbase)],
                        out_hbm.at[c, pl.ds(s * dbase, dbase)])

        @pl.when(s == 0)
        def _():
            pltpu.sync_copy(acc_sh.at[pl.ds(_NS * dbase, tail)],
                            out_hbm.at[c, pl.ds(_NS * dbase, tail)])

    return agg


def _make_query(n, h8, qpt):
    nq = qpt // _LANES
    mesh = plsc.VectorSubcoreMesh(core_axis_name="c", subcore_axis_name="s")

    @functools.partial(
        pl.kernel,
        out_type=(
            jax.ShapeDtypeStruct((_NW, 1, qpt), jnp.float32),
            jax.ShapeDtypeStruct((_NW, 1, qpt), jnp.float32),
        ),
        mesh=mesh,
        compiler_params=pltpu.CompilerParams(needs_layout_passes=False),
        scratch_types=[
            pltpu.VMEM((n * h8,), jnp.float32),
            pltpu.VMEM((1, qpt), jnp.int32),
            pltpu.VMEM((1, qpt), jnp.int32),
            pltpu.VMEM((1, qpt), jnp.float32),
            pltpu.VMEM((1, qpt), jnp.float32),
        ],
    )
    def qk(pq_hbm, q0_hbm, q1_hbm, z0_hbm, z1_hbm,
           pq_v, q0_v, q1_v, z0_v, z1_v):
        c = lax.axis_index("c")
        s = lax.axis_index("s")
        wid = c * _NS + s
        pltpu.sync_copy(pq_hbm, pq_v)
        pltpu.sync_copy(q0_hbm.at[wid], q0_v)
        pltpu.sync_copy(q1_hbm.at[wid], q1_v)

        def _chunk(i, _):
            q0 = q0_v[0, pl.ds(i * _LANES, _LANES)] * h8
            q1 = q1_v[0, pl.ds(i * _LANES, _LANES)] * h8
            a0 = plsc.load_gather(pq_v, [q0])
            a1 = plsc.load_gather(pq_v, [q0 + 1])
            b0 = plsc.load_gather(pq_v, [q1 + 2])
            b1 = plsc.load_gather(pq_v, [q1 + 3])
            z0_v[0, pl.ds(i * _LANES, _LANES)] = a0 + b0
            z1_v[0, pl.ds(i * _LANES, _LANES)] = a1 + b1
            return 0
        lax.fori_loop(0, nq, _chunk, 0)

        pltpu.sync_copy(z0_v, z0_hbm.at[wid])
        pltpu.sync_copy(z1_v, z1_hbm.at[wid])

    return qk


def kernel(x, edge_index, query_edges, edge_weight, W1, b1, W2, b2, Wlin,
           blin):
    n, f_in = x.shape
    e = edge_index.shape[1]
    q = query_edges.shape[0]
    h = W1.shape[1]
    ch = 80
    blk = 1000
    h8 = 8

    ept = -(-e // (_NW * 8 * ch)) * 8 * ch
    epad = _NW * ept - e

    qpt = -(-q // (_NW * _LANES)) * _LANES
    qpad = _NW * qpt - q

    spread = jnp.arange(epad, dtype=jnp.int32) % n
    srcp = jnp.concatenate([edge_index[0], spread]).reshape(_NW, ept // ch, ch)
    dstp = jnp.concatenate([edge_index[1], spread]).reshape(_NW, ept // ch, ch)
    wbits = jax.lax.bitcast_convert_type(
        jnp.pad(edge_weight, (0, epad)), jnp.int32).reshape(
        _NW, ept // ch, ch)
    edges3 = jnp.stack([srcp, dstp, wbits], axis=2)
    qp = jnp.pad(query_edges, ((0, qpad), (0, 0)))
    q0r = qp[:, 0].reshape(_NW, 1, qpt)
    q1r = qp[:, 1].reshape(_NW, 1, qpt)

    w8 = jnp.zeros((h, h8), jnp.float32)
    w8 = w8.at[:, 0:2].set(Wlin[:h])
    w8 = w8.at[:, 2:4].set(Wlin[h:])
    bl8 = jnp.zeros((1, h8), jnp.float32).at[0, 0:2].set(blin)

    agg = _make_agg(n, h, ept, ch)
    qk = _make_query(n, h8, qpt)

    h1 = _tc_mm1(x, W1, blk)
    p1 = agg(h1, edges3)
    h2 = _tc_mm2(p1[0], p1[1], b1.reshape(1, h), W2, blk)
    p2 = agg(h2, edges3)
    pq = _tc_mm3(p2[0], p2[1], b2.reshape(1, h), w8, bl8, blk)
    z0, z1 = qk(pq.reshape(-1), q0r, q1r)
    o0, o1 = _tc_log_softmax2(z0.reshape(_NW, qpt), z1.reshape(_NW, qpt))
    return jnp.stack([o0.reshape(-1)[:q], o1.reshape(-1)[:q]], axis=-1)


# speedup vs baseline: 1.1821x; 1.1821x over previous
import functools

import jax
import jax.numpy as jnp
from jax import lax
from jax.experimental import pallas as pl
from jax.experimental.pallas import tpu as pltpu
from jax.experimental.pallas import tpu_sc as plsc

_NC = 2
_NS = 16
_LANES = 16
_NW = _NC * _NS


def _mm1_body(x_ref, w_ref, o_ref):
    o_ref[...] = jnp.dot(x_ref[...], w_ref[...],
                         preferred_element_type=jnp.float32)


def _mm2_body(a0_ref, a1_ref, b_ref, w_ref, o_ref):
    h = jnp.maximum(a0_ref[...] + a1_ref[...] + b_ref[...], 0.0)
    o_ref[...] = jnp.dot(h, w_ref[...], preferred_element_type=jnp.float32)


def _mm3_body(a0_ref, a1_ref, b_ref, w_ref, bl_ref, o_ref):
    h = a0_ref[...] + a1_ref[...] + b_ref[...]
    o_ref[...] = (jnp.dot(h, w_ref[...], preferred_element_type=jnp.float32)
                  + bl_ref[...])


def _lsm_body(z0_ref, z1_ref, o0_ref, o1_ref):
    z0 = z0_ref[...]
    z1 = z1_ref[...]
    m = jnp.maximum(z0, z1)
    lse = m + jnp.log(jnp.exp(z0 - m) + jnp.exp(z1 - m))
    o0_ref[...] = z0 - lse
    o1_ref[...] = z1 - lse


def _tc_mm1(x, w, blk):
    n, f = x.shape
    h = w.shape[1]
    grid = n // blk
    return pl.pallas_call(
        _mm1_body,
        grid=(grid,),
        in_specs=[
            pl.BlockSpec((blk, f), lambda i: (i, 0)),
            pl.BlockSpec((f, h), lambda i: (0, 0)),
        ],
        out_specs=pl.BlockSpec((blk, h), lambda i: (i, 0)),
        out_shape=jax.ShapeDtypeStruct((n, h), jnp.float32),
    )(x, w)


def _tc_mm2(a0, a1, b_row, w, blk):
    n, f = a0.shape
    h = w.shape[1]
    grid = n // blk
    return pl.pallas_call(
        _mm2_body,
        grid=(grid,),
        in_specs=[
            pl.BlockSpec((blk, f), lambda i: (i, 0)),
            pl.BlockSpec((blk, f), lambda i: (i, 0)),
            pl.BlockSpec((1, f), lambda i: (0, 0)),
            pl.BlockSpec((f, h), lambda i: (0, 0)),
        ],
        out_specs=pl.BlockSpec((blk, h), lambda i: (i, 0)),
        out_shape=jax.ShapeDtypeStruct((n, h), jnp.float32),
    )(a0, a1, b_row, w)


def _tc_mm3(a0, a1, b_row, w8, bl8, blk):
    n, f = a0.shape
    h8 = w8.shape[1]
    grid = n // blk
    return pl.pallas_call(
        _mm3_body,
        grid=(grid,),
        in_specs=[
            pl.BlockSpec((blk, f), lambda i: (i, 0)),
            pl.BlockSpec((blk, f), lambda i: (i, 0)),
            pl.BlockSpec((1, f), lambda i: (0, 0)),
            pl.BlockSpec((f, h8), lambda i: (0, 0)),
            pl.BlockSpec((1, h8), lambda i: (0, 0)),
        ],
        out_specs=pl.BlockSpec((blk, h8), lambda i: (i, 0)),
        out_shape=jax.ShapeDtypeStruct((n, h8), jnp.float32),
    )(a0, a1, b_row, w8, bl8)


def _tc_log_softmax2(z0, z1):
    r, c = z0.shape
    return pl.pallas_call(
        _lsm_body,
        grid=(1,),
        in_specs=[
            pl.BlockSpec((r, c), lambda i: (0, 0)),
            pl.BlockSpec((r, c), lambda i: (0, 0)),
        ],
        out_specs=[
            pl.BlockSpec((r, c), lambda i: (0, 0)),
            pl.BlockSpec((r, c), lambda i: (0, 0)),
        ],
        out_shape=[
            jax.ShapeDtypeStruct((r, c), jnp.float32),
            jax.ShapeDtypeStruct((r, c), jnp.float32),
        ],
    )(z0, z1)


def _make_agg(n, f, ept, ch):
    nch = ept // ch
    noct = nch // 8
    dbase = (n // (8 * _NS)) * 8
    tail = n - _NS * dbase
    nzfull, zrem = divmod(dbase, ch)
    assert ch % _LANES == 0 and zrem % 8 == 0 and tail % 8 == 0 and tail <= ch
    assert nch == 8 * noct and noct >= 2
    nvec = f // _LANES
    mesh = plsc.VectorSubcoreMesh(core_axis_name="c", subcore_axis_name="s")

    @functools.partial(
        pl.kernel,
        out_type=jax.ShapeDtypeStruct((_NC, n, f), jnp.float32),
        mesh=mesh,
        compiler_params=pltpu.CompilerParams(needs_layout_passes=False),
        scratch_types=[
            [pltpu.VMEM((3, ch), jnp.int32)] * 8,
            [pltpu.VMEM((ch, f), jnp.float32)] * 4,
            pltpu.VMEM_SHARED((n, f), jnp.float32),
            [pltpu.SemaphoreType.DMA] * 8,
            [pltpu.SemaphoreType.DMA] * 4,
            [pltpu.SemaphoreType.DMA] * 4,
        ],
    )
    def agg(h_hbm, edges_hbm, out_hbm,
            idx_r, rows_r, acc_sh, sem_i, sem_g, sem_s):
        c = lax.axis_index("c")
        s = lax.axis_index("s")
        wid = c * _NS + s

        def _stage(k, u):
            pltpu.async_copy(edges_hbm.at[wid, k], idx_r[u], sem_i[u])

        def _wait_stage(u):
            pltpu.make_async_copy(edges_hbm.at[wid, 0], idx_r[u],
                                  sem_i[u]).wait()

        def _gather(u8, u4):
            pltpu.async_copy(h_hbm.at[idx_r[u8].at[0]], rows_r[u4],
                             sem_g[u4])

        def _wait_gather(u8, u4):
            pltpu.make_async_copy(h_hbm.at[idx_r[u8].at[0]], rows_r[u4],
                                  sem_g[u4]).wait()

        def _scale(u8, u4):
            w_v = idx_r[u8]
            rows_v = rows_r[u4]

            def _grp(g, _):
                base = g * _LANES
                wgrp = plsc.bitcast(w_v[2, pl.ds(base, _LANES)], jnp.float32)
                for j2 in range(_LANES):
                    wb = lax.gather(
                        wgrp,
                        jnp.full((_LANES, 1), j2, jnp.int32),
                        lax.GatherDimensionNumbers(
                            offset_dims=(), collapsed_slice_dims=(0,),
                            start_index_map=(0,)),
                        (1,),
                        mode=lax.GatherScatterMode.PROMISE_IN_BOUNDS)
                    for r in range(nvec):
                        rows_v[base + j2, pl.ds(r * _LANES, _LANES)] = (
                            rows_v[base + j2, pl.ds(r * _LANES, _LANES)] * wb)
                return 0
            lax.fori_loop(0, ch // _LANES, _grp, 0)

        def _scatter(u4, u8):
            pltpu.async_copy(rows_r[u4], acc_sh.at[idx_r[u8].at[1]],
                             sem_s[u4], add=True)

        def _wait_scatter(u4, u8):
            pltpu.make_async_copy(rows_r[u4], acc_sh.at[idx_r[u8].at[1]],
                                  sem_s[u4]).wait()

        def _zfill(i, _):
            for r in range(nvec):
                rows_r[0][i, pl.ds(r * _LANES, _LANES)] = jnp.zeros(
                    (_LANES,), jnp.float32)
            return 0
        lax.fori_loop(0, ch, _zfill, 0)
        for k in range(nzfull):
            pltpu.sync_copy(rows_r[0],
                            acc_sh.at[pl.ds(s * dbase + k * ch, ch)])
        if zrem:
            pltpu.sync_copy(rows_r[0].at[pl.ds(0, zrem)],
                            acc_sh.at[pl.ds(s * dbase + nzfull * ch, zrem)])

        @pl.when(s == 0)
        def _():
            pltpu.sync_copy(rows_r[0].at[pl.ds(0, tail)],
                            acc_sh.at[pl.ds(_NS * dbase, tail)])

        for k0 in range(4):
            _stage(k0, k0)
        plsc.subcore_barrier()
        _wait_stage(0)
        _gather(0, 0)
        _wait_stage(1)
        _gather(1, 1)

        def _oct(i, _):
            for kk in range(8):
                k = 8 * i + kk
                u4 = kk % 4
                u8 = kk
                g4 = (kk + 2) % 4
                g8 = (kk + 2) % 8
                s8 = (kk + 6) % 8
                p8 = (kk + 4) % 8
                _wait_gather(u8, u4)
                _scale(u8, u4)
                _scatter(u4, u8)
                if kk < 2:
                    @pl.when(i > 0)
                    def _():
                        _wait_scatter(g4, s8)
                else:
                    _wait_scatter(g4, s8)
                if kk >= 6:
                    @pl.when(i < noct - 1)
                    def _():
                        _wait_stage(g8)
                        _gather(g8, g4)
                else:
                    _wait_stage(g8)
                    _gather(g8, g4)
                if kk >= 4:
                    @pl.when(i < noct - 1)
                    def _():
                        _stage(k + 4, p8)
                else:
                    _stage(k + 4, p8)
            return 0
        lax.fori_loop(0, noct, _oct, 0)

        for k in (nch - 2, nch - 1):
            _wait_scatter(k % 4, k % 8)

        plsc.subcore_barrier()

        pltpu.sync_copy(acc_sh.at[pl.ds(s * dbase, dbase)],
                        out_hbm.at[c, pl.ds(s * dbase, dbase)])

        @pl.when(s == 0)
        def _():
            pltpu.sync_copy(acc_sh.at[pl.ds(_NS * dbase, tail)],
                            out_hbm.at[c, pl.ds(_NS * dbase, tail)])

    return agg


def _make_query(n, h8, qpt):
    nq = qpt // _LANES
    mesh = plsc.VectorSubcoreMesh(core_axis_name="c", subcore_axis_name="s")

    @functools.partial(
        pl.kernel,
        out_type=(
            jax.ShapeDtypeStruct((_NW, 1, qpt), jnp.float32),
            jax.ShapeDtypeStruct((_NW, 1, qpt), jnp.float32),
        ),
        mesh=mesh,
        compiler_params=pltpu.CompilerParams(needs_layout_passes=False),
        scratch_types=[
            pltpu.VMEM((n * h8,), jnp.float32),
            pltpu.VMEM((1, qpt), jnp.int32),
            pltpu.VMEM((1, qpt), jnp.int32),
            pltpu.VMEM((1, qpt), jnp.float32),
            pltpu.VMEM((1, qpt), jnp.float32),
        ],
    )
    def qk(pq_hbm, q0_hbm, q1_hbm, z0_hbm, z1_hbm,
           pq_v, q0_v, q1_v, z0_v, z1_v):
        c = lax.axis_index("c")
        s = lax.axis_index("s")
        wid = c * _NS + s
        pltpu.sync_copy(pq_hbm, pq_v)
        pltpu.sync_copy(q0_hbm.at[wid], q0_v)
        pltpu.sync_copy(q1_hbm.at[wid], q1_v)

        def _chunk(i, _):
            q0 = q0_v[0, pl.ds(i * _LANES, _LANES)] * h8
            q1 = q1_v[0, pl.ds(i * _LANES, _LANES)] * h8
            a0 = plsc.load_gather(pq_v, [q0])
            a1 = plsc.load_gather(pq_v, [q0 + 1])
            b0 = plsc.load_gather(pq_v, [q1 + 2])
            b1 = plsc.load_gather(pq_v, [q1 + 3])
            z0_v[0, pl.ds(i * _LANES, _LANES)] = a0 + b0
            z1_v[0, pl.ds(i * _LANES, _LANES)] = a1 + b1
            return 0
        lax.fori_loop(0, nq, _chunk, 0)

        pltpu.sync_copy(z0_v, z0_hbm.at[wid])
        pltpu.sync_copy(z1_v, z1_hbm.at[wid])

    return qk


def kernel(x, edge_index, query_edges, edge_weight, W1, b1, W2, b2, Wlin,
           blin):
    n, f_in = x.shape
    e = edge_index.shape[1]
    q = query_edges.shape[0]
    h = W1.shape[1]
    ch = 80
    blk = 1000
    h8 = 8

    ept = -(-e // (_NW * 8 * ch)) * 8 * ch
    epad = _NW * ept - e

    qpt = -(-q // (_NW * _LANES)) * _LANES
    qpad = _NW * qpt - q

    spread = jnp.arange(epad, dtype=jnp.int32) % n
    srcp = jnp.concatenate([edge_index[0], spread]).reshape(_NW, ept // ch, ch)
    dstp = jnp.concatenate([edge_index[1], spread]).reshape(_NW, ept // ch, ch)
    wbits = jax.lax.bitcast_convert_type(
        jnp.pad(edge_weight, (0, epad)), jnp.int32).reshape(
        _NW, ept // ch, ch)
    edges3 = jnp.stack([srcp, dstp, wbits], axis=2)
    qp = jnp.pad(query_edges, ((0, qpad), (0, 0)))
    q0r = qp[:, 0].reshape(_NW, 1, qpt)
    q1r = qp[:, 1].reshape(_NW, 1, qpt)

    w8 = jnp.zeros((h, h8), jnp.float32)
    w8 = w8.at[:, 0:2].set(Wlin[:h])
    w8 = w8.at[:, 2:4].set(Wlin[h:])
    bl8 = jnp.zeros((1, h8), jnp.float32).at[0, 0:2].set(blin)

    agg = _make_agg(n, h, ept, ch)
    qk = _make_query(n, h8, qpt)

    h1 = _tc_mm1(x, W1, blk)
    p1 = agg(h1, edges3)
    h2 = _tc_mm2(p1[0], p1[1], b1.reshape(1, h), W2, blk)
    p2 = agg(h2, edges3)
    pq = _tc_mm3(p2[0], p2[1], b2.reshape(1, h), w8, bl8, blk)
    z0, z1 = qk(pq.reshape(-1), q0r, q1r)
    o0, o1 = _tc_log_softmax2(z0.reshape(_NW, qpt), z1.reshape(_NW, qpt))
    return jnp.stack([o0.reshape(-1)[:q], o1.reshape(-1)[:q]], axis=-1)
